# baseline (speedup 1.0000x reference)
import jax
import jax.numpy as jnp
from jax import lax
from jax.experimental import pallas as pl
from jax.experimental.pallas import tpu as pltpu
from jax.experimental.pallas import tpu_sc as plsc

NSEG = 10000
ROWS = 320000
D = 128
NC = 2
NS = 16
DH = D // NC
B = 200
RPW = ROWS // NS
NBLK = RPW // B
NBUF = 7
RDA = 6
ZROWS = NSEG // NS
WB = NSEG // 10


def _sc_body(feats_hbm, ids_hbm, zeros_hbm, out_hbm, *scratch):
    feats_bufs = scratch[0:NBUF]
    ids_bufs = scratch[NBUF:2 * NBUF]
    sems_f = scratch[2 * NBUF:3 * NBUF]
    sems_i = scratch[3 * NBUF:4 * NBUF]
    sems_s = scratch[4 * NBUF:5 * NBUF]
    acc = scratch[5 * NBUF]

    c = lax.axis_index("c")
    s = lax.axis_index("s")

    def start_read(b, slot):
        gb = s * NBLK + b
        row0 = gb * B
        cf = pltpu.async_copy(
            feats_hbm.at[pl.ds(row0, B), pl.ds(c * DH, DH)],
            feats_bufs[slot], sems_f[slot])
        ci = pltpu.async_copy(ids_hbm.at[gb], ids_bufs[slot], sems_i[slot])
        return cf, ci

    reads = {}
    for p in range(RDA):
        reads[p] = start_read(p, p % NBUF)

    pltpu.sync_copy(zeros_hbm, acc.at[pl.ds(s * ZROWS, ZROWS)])
    plsc.subcore_barrier()

    scats = {}
    for b in range(NBLK):
        slot = b % NBUF
        cf, ci = reads.pop(b)
        cf.wait()
        ci.wait()
        scats[b] = pltpu.async_copy(
            feats_bufs[slot], acc.at[ids_bufs[slot]], sems_s[slot], add=True)
        nb = b + RDA
        if nb < NBLK:
            nslot = nb % NBUF
            prev = nb - NBUF
            if prev >= 0:
                scats.pop(prev).wait()
            reads[nb] = start_read(nb, nslot)
    for b in sorted(scats):
        scats[b].wait()
    plsc.subcore_barrier()

    @pl.when(s < 10)
    def _():
        pltpu.sync_copy(
            acc.at[pl.ds(s * WB, WB)],
            out_hbm.at[pl.ds(s * WB, WB), pl.ds(c * DH, DH)],
        )


@jax.jit
def _pool_sum(feats, ids3, zeros):
    mesh = plsc.VectorSubcoreMesh(
        core_axis_name="c", subcore_axis_name="s", num_cores=NC, num_subcores=NS
    )
    return pl.kernel(
        _sc_body,
        out_type=jax.ShapeDtypeStruct((NSEG, D), jnp.float32),
        mesh=mesh,
        scratch_types=(
            [pltpu.VMEM((B, DH), jnp.float32) for _ in range(NBUF)]
            + [pltpu.VMEM((B,), jnp.int32) for _ in range(NBUF)]
            + [pltpu.SemaphoreType.DMA for _ in range(3 * NBUF)]
            + [pltpu.VMEM_SHARED((NSEG, DH), jnp.float32)]
        ),
        compiler_params=pltpu.CompilerParams(use_tc_tiling_on_sc=False),
    )(feats, ids3, zeros)


def kernel(feats, batch):
    ids3 = batch.astype(jnp.int32).reshape(ROWS // B, B)
    zeros = jnp.zeros((ZROWS, DH), jnp.float32)
    return _pool_sum(feats, ids3, zeros)


# speedup vs baseline: 1.0050x; 1.0050x over previous
import jax
import jax.numpy as jnp
from jax import lax
from jax.experimental import pallas as pl
from jax.experimental.pallas import tpu as pltpu
from jax.experimental.pallas import tpu_sc as plsc

NSEG = 10000
ROWS = 320000
D = 128
NC = 2
NS = 16
DH = D // NC
B = 160
RPW = ROWS // NS
NBLK = RPW // B
NBUF = 8
RDA = 6
ZROWS = NSEG // NS
WB = NSEG // 10


def _sc_body(feats_hbm, ids_hbm, zeros_hbm, out_hbm, *scratch):
    feats_bufs = scratch[0:NBUF]
    ids_bufs = scratch[NBUF:2 * NBUF]
    sems_f = scratch[2 * NBUF:3 * NBUF]
    sems_i = scratch[3 * NBUF:4 * NBUF]
    sems_s = scratch[4 * NBUF:5 * NBUF]
    acc = scratch[5 * NBUF]

    c = lax.axis_index("c")
    s = lax.axis_index("s")

    def start_read(b, slot):
        gb = s * NBLK + b
        row0 = gb * B
        cf = pltpu.async_copy(
            feats_hbm.at[pl.ds(row0, B), pl.ds(c * DH, DH)],
            feats_bufs[slot], sems_f[slot])
        ci = pltpu.async_copy(ids_hbm.at[gb], ids_bufs[slot], sems_i[slot])
        return cf, ci

    reads = {}
    for p in range(RDA):
        reads[p] = start_read(p, p % NBUF)

    pltpu.sync_copy(zeros_hbm, acc.at[pl.ds(s * ZROWS, ZROWS)])
    plsc.subcore_barrier()

    scats = {}
    for b in range(NBLK):
        slot = b % NBUF
        cf, ci = reads.pop(b)
        cf.wait()
        ci.wait()
        scats[b] = pltpu.async_copy(
            feats_bufs[slot], acc.at[ids_bufs[slot]], sems_s[slot], add=True)
        nb = b + RDA
        if nb < NBLK:
            nslot = nb % NBUF
            prev = nb - NBUF
            if prev >= 0:
                scats.pop(prev).wait()
            reads[nb] = start_read(nb, nslot)
    for b in sorted(scats):
        scats[b].wait()
    plsc.subcore_barrier()

    @pl.when(s < 10)
    def _():
        pltpu.sync_copy(
            acc.at[pl.ds(s * WB, WB)],
            out_hbm.at[pl.ds(s * WB, WB), pl.ds(c * DH, DH)],
        )


@jax.jit
def _pool_sum(feats, ids3, zeros):
    mesh = plsc.VectorSubcoreMesh(
        core_axis_name="c", subcore_axis_name="s", num_cores=NC, num_subcores=NS
    )
    return pl.kernel(
        _sc_body,
        out_type=jax.ShapeDtypeStruct((NSEG, D), jnp.float32),
        mesh=mesh,
        scratch_types=(
            [pltpu.VMEM((B, DH), jnp.float32) for _ in range(NBUF)]
            + [pltpu.VMEM((B,), jnp.int32) for _ in range(NBUF)]
            + [pltpu.SemaphoreType.DMA for _ in range(3 * NBUF)]
            + [pltpu.VMEM_SHARED((NSEG, DH), jnp.float32)]
        ),
        compiler_params=pltpu.CompilerParams(use_tc_tiling_on_sc=False),
    )(feats, ids3, zeros)


def kernel(feats, batch):
    ids3 = batch.astype(jnp.int32).reshape(ROWS // B, B)
    zeros = jnp.zeros((ZROWS, DH), jnp.float32)
    return _pool_sum(feats, ids3, zeros)


# speedup vs baseline: 1.0135x; 1.0085x over previous
import jax
import jax.numpy as jnp
from jax import lax
from jax.experimental import pallas as pl
from jax.experimental.pallas import tpu as pltpu
from jax.experimental.pallas import tpu_sc as plsc

NSEG = 10000
ROWS = 320000
D = 128
NC = 2
NS = 16
DH = D // NC
B = 200
RPW = ROWS // NS
NBLK = RPW // B
NBUF = 7
RDA = 5
ZROWS = NSEG // NS
WB = NSEG // 10


def _sc_body(feats_hbm, ids_hbm, zeros_hbm, out_hbm, *scratch):
    feats_bufs = scratch[0:NBUF]
    ids_bufs = scratch[NBUF:2 * NBUF]
    sems_f = scratch[2 * NBUF:3 * NBUF]
    sems_i = scratch[3 * NBUF:4 * NBUF]
    sems_s = scratch[4 * NBUF:5 * NBUF]
    acc = scratch[5 * NBUF]

    c = lax.axis_index("c")
    s = lax.axis_index("s")

    def start_read(b, slot):
        gb = s * NBLK + b
        row0 = gb * B
        cf = pltpu.async_copy(
            feats_hbm.at[pl.ds(row0, B), pl.ds(c * DH, DH)],
            feats_bufs[slot], sems_f[slot])
        ci = pltpu.async_copy(ids_hbm.at[gb], ids_bufs[slot], sems_i[slot])
        return cf, ci

    reads = {}
    for p in range(RDA):
        reads[p] = start_read(p, p % NBUF)

    pltpu.sync_copy(zeros_hbm, acc.at[pl.ds(s * ZROWS, ZROWS)])
    plsc.subcore_barrier()

    scats = {}
    for b in range(NBLK):
        slot = b % NBUF
        cf, ci = reads.pop(b)
        cf.wait()
        ci.wait()
        scats[b] = pltpu.async_copy(
            feats_bufs[slot], acc.at[ids_bufs[slot]], sems_s[slot], add=True)
        nb = b + RDA
        if nb < NBLK:
            nslot = nb % NBUF
            prev = nb - NBUF
            if prev >= 0:
                scats.pop(prev).wait()
            reads[nb] = start_read(nb, nslot)
    for b in sorted(scats):
        scats[b].wait()
    plsc.subcore_barrier()

    @pl.when(s < 10)
    def _():
        pltpu.sync_copy(
            acc.at[pl.ds(s * WB, WB)],
            out_hbm.at[pl.ds(s * WB, WB), pl.ds(c * DH, DH)],
        )


@jax.jit
def _pool_sum(feats, ids3, zeros):
    mesh = plsc.VectorSubcoreMesh(
        core_axis_name="c", subcore_axis_name="s", num_cores=NC, num_subcores=NS
    )
    return pl.kernel(
        _sc_body,
        out_type=jax.ShapeDtypeStruct((NSEG, D), jnp.float32),
        mesh=mesh,
        scratch_types=(
            [pltpu.VMEM((B, DH), jnp.float32) for _ in range(NBUF)]
            + [pltpu.VMEM((B,), jnp.int32) for _ in range(NBUF)]
            + [pltpu.SemaphoreType.DMA for _ in range(3 * NBUF)]
            + [pltpu.VMEM_SHARED((NSEG, DH), jnp.float32)]
        ),
        compiler_params=pltpu.CompilerParams(use_tc_tiling_on_sc=False),
    )(feats, ids3, zeros)


def kernel(feats, batch):
    ids3 = batch.astype(jnp.int32).reshape(ROWS // B, B)
    zeros = jnp.zeros((ZROWS, DH), jnp.float32)
    return _pool_sum(feats, ids3, zeros)
